# baseline (speedup 1.0000x reference)
import functools

import jax
import jax.numpy as jnp
from jax import lax
from jax.experimental import pallas as pl
from jax.experimental.pallas import tpu as pltpu
from jax.experimental.pallas import tpu_sc as plsc

BATCH = 16384
FACTORS = 64
NC = 2
NS = 16
NW = NC * NS
BPW = BATCH // NW
LANE = 128
FLUSH = 64
RING = 12


def _gather_body(su_hbm, so_hbm, si_hbm, to_hbm, eu_hbm, ei_hbm,
                 a_hbm, b_hbm,
                 vals, pos, slabs, rowbuf, meta, meta_ul, sem, sem2):
    wid = lax.axis_index("s") * NC + lax.axis_index("c")
    base = wid * BPW

    for sv_hbm, sp_hbm, tab, dst in (
        (su_hbm, so_hbm, eu_hbm, a_hbm),
        (si_hbm, to_hbm, ei_hbm, b_hbm),
    ):
        pltpu.sync_copy(sp_hbm.at[pl.ds(base, BPW)], pos)
        gcopies = [
            pltpu.async_copy(
                sv_hbm.at[pos.at[pl.ds(j * 128, 128)]],
                vals.at[pl.ds(j * 128, 128)], sem2)
            for j in range(BPW // 128)
        ]
        for g in gcopies:
            g.wait()

        def passa(ch, carry):
            prev, cnt = carry
            coff = pl.multiple_of(ch * 16, 8)
            vvec = vals[pl.ds(coff, 16)]
            for t in range(16):
                u = vvec[t]
                ut = lax.shift_right_logical(u, 7)
                meta_ul[ch * 16 + t] = lax.bitwise_and(u, LANE - 1)
                isnew = ut != prev

                @pl.when(isnew)
                def _(ut=ut, t=t, ch=ch, cnt=cnt):
                    meta[cnt] = ut * 1024 + ch * 16 + t

                cnt = cnt + isnew.astype(jnp.int32)
                prev = ut
            return prev, cnt

        _, nb = lax.fori_loop(
            0, BPW // 16, passa, (jnp.int32(-1), jnp.int32(0))
        )
        meta[nb] = jnp.int32(BPW)

        def fetch(rr, buf, tab=tab):
            ut = lax.shift_right_logical(meta[rr], 10)
            off = pl.multiple_of(ut * LANE, LANE)
            pltpu.async_copy(tab.at[:, pl.ds(off, LANE)], slabs.at[buf], sem)

        for kk in range(RING - 1):
            fetch(jnp.minimum(jnp.int32(kk), nb - 1), jnp.int32(kk))

        def run_cond(st):
            return st[0] < nb

        def run_body(st, tab=tab, dst=dst):
            r, ptr = st
            fetch(jnp.minimum(r + RING - 1, nb - 1), lax.rem(r + RING - 1, RING))
            pltpu.make_async_copy(
                tab.at[:, pl.ds(0, LANE)], slabs.at[lax.rem(r, RING)], sem
            ).wait()
            bufv = lax.rem(r, RING)
            end = lax.bitwise_and(meta[r + 1], 1023)

            def lk_cond(p):
                return p < end

            def lk_body(p, dst=dst):
                ul = meta_ul[p]
                row = lax.bitwise_and(p, FLUSH - 1)
                for c in range(FACTORS // 16):
                    g = plsc.load_gather(
                        slabs,
                        [jnp.full((16,), bufv, jnp.int32),
                         c * 16 + lax.iota(jnp.int32, 16),
                         jnp.full((16,), ul, jnp.int32)],
                    )
                    rowbuf[row, pl.ds(c * 16, 16)] = g

                @pl.when(row == FLUSH - 1)
                def _(dst=dst):
                    blk = pl.multiple_of(
                        lax.bitwise_and(p, ~(FLUSH - 1)), 8
                    )
                    pltpu.async_copy(
                        rowbuf, dst.at[pos.at[pl.ds(blk, FLUSH)]], sem2
                    ).wait()

                return p + 1

            ptr = lax.while_loop(lk_cond, lk_body, ptr)
            return r + 1, ptr

        lax.while_loop(run_cond, run_body, (jnp.int32(0), jnp.int32(0)))

        for kk in range(RING - 1):
            pltpu.make_async_copy(
                tab.at[:, pl.ds(0, LANE)], slabs.at[kk], sem
            ).wait()


def _dot_body(a_hbm, b_hbm, out_hbm, arows, brows, outv, sem):
    wid = lax.axis_index("s") * NC + lax.axis_index("c")
    base = wid * BPW
    lane = lax.iota(jnp.int32, 16)

    for blk in range(BPW // 128):
        pltpu.sync_copy(a_hbm.at[pl.ds(base + blk * 128, 128), :], arows)
        pltpu.sync_copy(b_hbm.at[pl.ds(base + blk * 128, 128), :], brows)

        def group(g, carry, blk=blk):
            sums = jnp.zeros((16,), jnp.float32)
            for t in range(16):
                r = g * 16 + t
                acc = arows[r, pl.ds(0, 16)] * brows[r, pl.ds(0, 16)]
                acc = acc + arows[r, pl.ds(16, 16)] * brows[r, pl.ds(16, 16)]
                acc = acc + arows[r, pl.ds(32, 16)] * brows[r, pl.ds(32, 16)]
                acc = acc + arows[r, pl.ds(48, 16)] * brows[r, pl.ds(48, 16)]
                sums = jnp.where(lane == t, jnp.sum(acc, axis=0), sums)
            outv[pl.ds(blk * 128 + g * 16, 16)] = sums
            return carry

        lax.fori_loop(0, 128 // 16, group, 0)

    pltpu.sync_copy(outv, out_hbm.at[pl.ds(base, BPW)])


@jax.jit
def kernel(user, item, embed_user, embed_item):
    user = user.astype(jnp.int32)
    item = item.astype(jnp.int32)
    so = jnp.argsort(user).astype(jnp.int32)
    to = jnp.argsort(item).astype(jnp.int32)

    mesh = plsc.VectorSubcoreMesh(core_axis_name="c", subcore_axis_name="s")
    params = pltpu.CompilerParams(
        needs_layout_passes=False, use_tc_tiling_on_sc=True
    )

    gather = pl.kernel(
        _gather_body,
        out_type=(
            jax.ShapeDtypeStruct((BATCH, LANE), jnp.float32),
            jax.ShapeDtypeStruct((BATCH, LANE), jnp.float32),
        ),
        mesh=mesh,
        scratch_types=[
            pltpu.VMEM((BPW,), jnp.int32),
            pltpu.VMEM((BPW,), jnp.int32),
            pltpu.VMEM((RING, FACTORS, LANE), jnp.float32),
            pltpu.VMEM((FLUSH, LANE), jnp.float32),
            pltpu.SMEM((BPW + 2,), jnp.int32),
            pltpu.SMEM((BPW,), jnp.int32),
            pltpu.SemaphoreType.DMA,
            pltpu.SemaphoreType.DMA,
        ],
        compiler_params=params,
    )
    a, b = gather(user, so, item, to, embed_user.T, embed_item.T)

    dot = pl.kernel(
        _dot_body,
        out_type=jax.ShapeDtypeStruct((BATCH,), jnp.float32),
        mesh=mesh,
        scratch_types=[
            pltpu.VMEM((128, LANE), jnp.float32),
            pltpu.VMEM((128, LANE), jnp.float32),
            pltpu.VMEM((BPW,), jnp.float32),
            pltpu.SemaphoreType.DMA,
        ],
        compiler_params=params,
    )
    return dot(a, b)


# speedup vs baseline: 1.0341x; 1.0341x over previous
import functools

import jax
import jax.numpy as jnp
from jax import lax
from jax.experimental import pallas as pl
from jax.experimental.pallas import tpu as pltpu
from jax.experimental.pallas import tpu_sc as plsc

BATCH = 16384
FACTORS = 64
NC = 2
NS = 16
NW = NC * NS
BPW = BATCH // NW
LANE = 128
FLUSH = 64
RING = 8


def _gather_body(su_hbm, so_hbm, si_hbm, to_hbm, eu_hbm, ei_hbm,
                 a_hbm, b_hbm,
                 vals, pos, slabs, rowbuf, meta, meta_ul, sem, sem2):
    wid = lax.axis_index("s") * NC + lax.axis_index("c")
    base = wid * BPW

    for sv_hbm, sp_hbm, tab, dst in (
        (su_hbm, so_hbm, eu_hbm, a_hbm),
        (si_hbm, to_hbm, ei_hbm, b_hbm),
    ):
        pltpu.sync_copy(sp_hbm.at[pl.ds(base, BPW)], pos)
        gcopies = [
            pltpu.async_copy(
                sv_hbm.at[pos.at[pl.ds(j * 128, 128)]],
                vals.at[pl.ds(j * 128, 128)], sem2)
            for j in range(BPW // 128)
        ]
        for g in gcopies:
            g.wait()

        def passa(ch, carry):
            prev, cnt = carry
            coff = pl.multiple_of(ch * 16, 8)
            vvec = vals[pl.ds(coff, 16)]
            for t in range(16):
                u = vvec[t]
                ut = lax.shift_right_logical(u, 7)
                meta_ul[ch * 16 + t] = lax.bitwise_and(u, LANE - 1)
                isnew = ut != prev

                @pl.when(isnew)
                def _(ut=ut, t=t, ch=ch, cnt=cnt):
                    meta[cnt] = ut * 1024 + ch * 16 + t

                cnt = cnt + isnew.astype(jnp.int32)
                prev = ut
            return prev, cnt

        _, nb = lax.fori_loop(
            0, BPW // 16, passa, (jnp.int32(-1), jnp.int32(0))
        )
        meta[nb] = jnp.int32(BPW)

        def fetch(rr, buf, tab=tab):
            ut = lax.shift_right_logical(meta[rr], 10)
            off = pl.multiple_of(ut * LANE, LANE)
            pltpu.async_copy(tab.at[:, pl.ds(off, LANE)], slabs.at[buf], sem)

        for kk in range(RING - 1):
            fetch(jnp.minimum(jnp.int32(kk), nb - 1), jnp.int32(kk))

        def run_cond(st):
            return st[0] < nb

        def run_body(st, tab=tab, dst=dst):
            r, ptr = st
            fetch(jnp.minimum(r + RING - 1, nb - 1), lax.rem(r + RING - 1, RING))
            pltpu.make_async_copy(
                tab.at[:, pl.ds(0, LANE)], slabs.at[lax.rem(r, RING)], sem
            ).wait()
            bufv = lax.rem(r, RING)
            end = lax.bitwise_and(meta[r + 1], 1023)

            def lk_cond(p):
                return p < end

            def lk_body(p, dst=dst):
                ul = meta_ul[p]
                row = lax.bitwise_and(p, FLUSH - 1)
                par = lax.bitwise_and(lax.shift_right_logical(p, 6), 1)
                for c in range(FACTORS // 16):
                    g = plsc.load_gather(
                        slabs,
                        [jnp.full((16,), bufv, jnp.int32),
                         c * 16 + lax.iota(jnp.int32, 16),
                         jnp.full((16,), ul, jnp.int32)],
                    )
                    rowbuf[par, row, pl.ds(c * 16, 16)] = g

                @pl.when(row == FLUSH - 1)
                def _(dst=dst, par=par):
                    blk = pl.multiple_of(
                        lax.bitwise_and(p, ~(FLUSH - 1)), 8
                    )

                    @pl.when(blk >= FLUSH)
                    def _():
                        pltpu.make_async_copy(
                            a_hbm.at[pl.ds(0, FLUSH), :], rowbuf.at[0], sem2
                        ).wait()

                    pltpu.async_copy(
                        rowbuf.at[par], dst.at[pos.at[pl.ds(blk, FLUSH)]],
                        sem2,
                    )

                return p + 1

            ptr = lax.while_loop(lk_cond, lk_body, ptr)
            return r + 1, ptr

        lax.while_loop(run_cond, run_body, (jnp.int32(0), jnp.int32(0)))

        pltpu.make_async_copy(
            a_hbm.at[pl.ds(0, FLUSH), :], rowbuf.at[0], sem2
        ).wait()
        for kk in range(RING - 1):
            pltpu.make_async_copy(
                tab.at[:, pl.ds(0, LANE)], slabs.at[kk], sem
            ).wait()


def _dot_body(a_hbm, b_hbm, out_hbm, arows, brows, outv, sem):
    wid = lax.axis_index("s") * NC + lax.axis_index("c")
    base = wid * BPW
    lane = lax.iota(jnp.int32, 16)

    for blk in range(BPW // 128):
        pltpu.sync_copy(a_hbm.at[pl.ds(base + blk * 128, 128), :], arows)
        pltpu.sync_copy(b_hbm.at[pl.ds(base + blk * 128, 128), :], brows)

        def group(g, carry, blk=blk):
            sums = jnp.zeros((16,), jnp.float32)
            for t in range(16):
                r = g * 16 + t
                acc = arows[r, pl.ds(0, 16)] * brows[r, pl.ds(0, 16)]
                acc = acc + arows[r, pl.ds(16, 16)] * brows[r, pl.ds(16, 16)]
                acc = acc + arows[r, pl.ds(32, 16)] * brows[r, pl.ds(32, 16)]
                acc = acc + arows[r, pl.ds(48, 16)] * brows[r, pl.ds(48, 16)]
                sums = jnp.where(lane == t, jnp.sum(acc, axis=0), sums)
            outv[pl.ds(blk * 128 + g * 16, 16)] = sums
            return carry

        lax.fori_loop(0, 128 // 16, group, 0)

    pltpu.sync_copy(outv, out_hbm.at[pl.ds(base, BPW)])


@jax.jit
def kernel(user, item, embed_user, embed_item):
    user = user.astype(jnp.int32)
    item = item.astype(jnp.int32)
    so = jnp.argsort(user).astype(jnp.int32)
    to = jnp.argsort(item).astype(jnp.int32)

    mesh = plsc.VectorSubcoreMesh(core_axis_name="c", subcore_axis_name="s")
    params = pltpu.CompilerParams(
        needs_layout_passes=False, use_tc_tiling_on_sc=True
    )

    gather = pl.kernel(
        _gather_body,
        out_type=(
            jax.ShapeDtypeStruct((BATCH, LANE), jnp.float32),
            jax.ShapeDtypeStruct((BATCH, LANE), jnp.float32),
        ),
        mesh=mesh,
        scratch_types=[
            pltpu.VMEM((BPW,), jnp.int32),
            pltpu.VMEM((BPW,), jnp.int32),
            pltpu.VMEM((RING, FACTORS, LANE), jnp.float32),
            pltpu.VMEM((2, FLUSH, LANE), jnp.float32),
            pltpu.SMEM((BPW + 2,), jnp.int32),
            pltpu.SMEM((BPW,), jnp.int32),
            pltpu.SemaphoreType.DMA,
            pltpu.SemaphoreType.DMA,
        ],
        compiler_params=params,
    )
    a, b = gather(user, so, item, to, embed_user.T, embed_item.T)

    dot = pl.kernel(
        _dot_body,
        out_type=jax.ShapeDtypeStruct((BATCH,), jnp.float32),
        mesh=mesh,
        scratch_types=[
            pltpu.VMEM((128, LANE), jnp.float32),
            pltpu.VMEM((128, LANE), jnp.float32),
            pltpu.VMEM((BPW,), jnp.float32),
            pltpu.SemaphoreType.DMA,
        ],
        compiler_params=params,
    )
    return dot(a, b)
